# Initial kernel scaffold; baseline (speedup 1.0000x reference)
import jax
import jax.numpy as jnp
from jax.experimental import pallas as pl


def kernel(locs, W_init, b_init, eW0, eb0, eW1, eb1, eW2, eb2, nW0, nb0, nW1, nb1, nW2, nb2):
    raise NotImplementedError("write your pallas kernel here")


# speedup vs baseline: 25.5239x; 25.5239x over previous
import jax
import jax.numpy as jnp
from jax.experimental import pallas as pl
from jax.experimental.pallas import tpu as pltpu

B, N, D = 4, 256, 128
H1, H2 = 64, 32
L = 3
TI = 32
NB = N // TI

_INTERPRET = False


def _dot(a, b):
    return jnp.dot(a, b, preferred_element_type=jnp.float32)


def _bf(x):
    return x.astype(jnp.bfloat16).astype(jnp.float32)


def _body(locs_ref, Wi_ref, bi_ref,
          eW0s_ref, eW0d_ref, eW0e_ref, eb0_ref, eW1_ref, eb1_ref,
          eW2r_ref, eb2_ref,
          nW0x_ref, nW0a_ref, nb0_ref, nW1_ref, nb1_ref, nW2_ref, nb2_ref,
          out_nf_ref, out_h_ref,
          efs_ref, uefs_ref, As_ref, Bs_ref):
    f32 = jnp.float32
    locs = locs_ref[0]
    Wi = Wi_ref[:]
    h = _dot(locs, Wi) + bi_ref[0]
    out_h_ref[0] = h

    def ef_tile(ib, carry):
        off = ib * TI
        ht = out_h_ref[0, pl.ds(off, TI), :]
        diff = ht[:, None, :] - h[None, :, :]
        efs_ref[pl.ds(off, TI), :] = jnp.sqrt(
            jnp.sum(diff * diff, axis=-1))
        return carry

    jax.lax.fori_loop(0, NB, ef_tile, 0)

    ones_col = jnp.ones((N, 1), dtype=f32)
    nf = h
    for l in range(L):
        src_ref = efs_ref if l % 2 == 0 else uefs_ref
        dst_ref = uefs_ref if l % 2 == 0 else efs_ref
        As_ref[...] = _dot(nf, eW0s_ref[l])
        Bs_ref[...] = _dot(nf, eW0d_ref[l])
        w0e = eW0e_ref[l]
        b0 = eb0_ref[l]
        W1 = eW1_ref[l]
        b1 = eb1_ref[l]
        w2 = eW2r_ref[l]
        b2 = eb2_ref[l]

        def tile_body(ib, carry):
            off = ib * TI
            a = As_ref[pl.ds(off, TI), :]
            e = src_ref[pl.ds(off, TI), :]
            z = (a[:, None, :] + Bs_ref[...][None, :, :]
                 + _bf(e)[:, :, None] * _bf(w0e) + b0)
            h0 = jnp.maximum(z, 0.0)
            h1 = jnp.maximum(
                _dot(h0.reshape(TI * N, H1), W1) + b1, 0.0)
            u = jnp.sum(_bf(h1.reshape(TI, N, H2)) * _bf(w2),
                        axis=-1) + b2
            dst_ref[pl.ds(off, TI), :] = u
            return carry

        jax.lax.fori_loop(0, NB, tile_body, 0)
        uef = dst_ref[...]
        aggr = jax.lax.dot_general(uef, ones_col, (((0,), (0,)), ((), ())),
                                   preferred_element_type=f32,
                                   precision=jax.lax.Precision.HIGHEST)
        zn = (_dot(nf, nW0x_ref[l]) + _bf(aggr) * _bf(nW0a_ref[l])
              + nb0_ref[l])
        n0 = jnp.maximum(zn, 0.0)
        n1 = jnp.maximum(_dot(n0, nW1_ref[l]) + nb1_ref[l], 0.0)
        nf = nf + _dot(n1, nW2_ref[l]) + nb2_ref[l]
    out_nf_ref[0] = nf


def _full(shape):
    nd = len(shape)
    return pl.BlockSpec(shape, lambda b: (0,) * nd)


@jax.jit
def kernel(locs, W_init, b_init, eW0, eb0, eW1, eb1, eW2, eb2,
           nW0, nb0, nW1, nb1, nW2, nb2):
    eW0s = eW0[:, :D, :]
    eW0d = eW0[:, D:2 * D, :]
    eW0e = eW0[:, 2 * D, :]
    eW2r = eW2[:, :, 0]
    nW0x = nW0[:, :D, :]
    nW0a = nW0[:, D, :]
    bi = b_init.reshape(1, D)

    args = (locs, W_init, bi, eW0s, eW0d, eW0e, eb0, eW1, eb1, eW2r, eb2,
            nW0x, nW0a, nb0, nW1, nb1, nW2, nb2)
    in_specs = [pl.BlockSpec((1, N, 2), lambda b: (b, 0, 0))]
    in_specs += [_full(a.shape) for a in args[1:]]
    out_nf, out_h = pl.pallas_call(
        _body,
        grid=(B,),
        in_specs=in_specs,
        out_specs=[pl.BlockSpec((1, N, D), lambda b: (b, 0, 0))] * 2,
        out_shape=[jax.ShapeDtypeStruct((B, N, D), jnp.float32)] * 2,
        scratch_shapes=[
            pltpu.VMEM((N, N), jnp.float32),
            pltpu.VMEM((N, N), jnp.float32),
            pltpu.VMEM((N, H1), jnp.float32),
            pltpu.VMEM((N, H1), jnp.float32),
        ],
        interpret=_INTERPRET,
    )(*args)
    return out_nf, out_h


# speedup vs baseline: 34.7332x; 1.3608x over previous
import jax
import jax.numpy as jnp
from jax.experimental import pallas as pl
from jax.experimental.pallas import tpu as pltpu

B, N, D = 4, 256, 128
H1, H2 = 64, 32
L = 3
TI = 32
NB = N // TI

_INTERPRET = False


def _dot(a, b):
    return jnp.dot(a, b, preferred_element_type=jnp.float32)


def _hdot(a, b):
    return jnp.dot(a, b, preferred_element_type=jnp.float32,
                   precision=jax.lax.Precision.HIGHEST)


def _bf(x):
    return x.astype(jnp.bfloat16).astype(jnp.float32)


def _body(locs_ref, Wi_ref, bi_ref,
          eW0s_ref, eW0d_ref, eW0e_ref, eb0_ref, eW1_ref, eb1_ref,
          eW2r_ref, eb2_ref,
          nW0x_ref, nW0a_ref, nb0_ref, nW1_ref, nb1_ref, nW2_ref, nb2_ref,
          out_nf_ref, out_h_ref,
          efs_ref, uefs_ref, As_ref, Bs_ref):
    f32 = jnp.float32
    locs = locs_ref[0]
    Wi = Wi_ref[:]
    h = _dot(locs, Wi) + bi_ref[0]
    out_h_ref[0] = h

    G = jax.lax.dot_general(h, h, (((1,), (1,)), ((), ())),
                            preferred_element_type=f32,
                            precision=jax.lax.Precision.HIGHEST)
    ii = jax.lax.broadcasted_iota(jnp.int32, (N, N), 0)
    jj = jax.lax.broadcasted_iota(jnp.int32, (N, N), 1)
    eye = (ii == jj).astype(f32)
    Gd = G * eye
    sqi = jnp.sum(Gd, axis=1, keepdims=True)
    sqj = jnp.sum(Gd, axis=0, keepdims=True)
    efs_ref[...] = jnp.sqrt(jnp.maximum(sqi + sqj - 2.0 * G, 0.0))

    ones_col = jnp.ones((N, 1), dtype=f32)
    nf = h
    for l in range(L):
        src_ref = efs_ref if l % 2 == 0 else uefs_ref
        dst_ref = uefs_ref if l % 2 == 0 else efs_ref
        As_ref[...] = _dot(nf, eW0s_ref[l]) + eb0_ref[l]
        Bs_ref[...] = _dot(nf, eW0d_ref[l])
        w0e = eW0e_ref[l]
        W1 = eW1_ref[l]
        b1 = eb1_ref[l]
        w2 = eW2r_ref[l]
        b2 = eb2_ref[l]
        last = (l == L - 1)

        def tile_body(ib, S):
            off = ib * TI
            a = As_ref[pl.ds(off, TI), :]
            e = src_ref[pl.ds(off, TI), :]
            z = (a[:, None, :] + Bs_ref[...][None, :, :]
                 + _bf(e)[:, :, None] * _bf(w0e))
            h0 = jnp.maximum(z, 0.0)
            h1 = jnp.maximum(
                _dot(h0.reshape(TI * N, H1), W1) + b1, 0.0)
            h1t = _bf(h1.reshape(TI, N, H2))
            if last:
                return S + jnp.sum(h1t, axis=0)
            u = jnp.sum(h1t * _bf(w2), axis=-1) + b2
            dst_ref[pl.ds(off, TI), :] = u
            return S

        S0 = jnp.zeros((N, H2), dtype=f32)
        S = jax.lax.fori_loop(0, NB, tile_body, S0)
        if last:
            aggr = _hdot(S, _bf(w2)[:, None]) + jnp.float32(N) * b2
        else:
            uef = dst_ref[...]
            aggr = jax.lax.dot_general(
                uef, ones_col, (((0,), (0,)), ((), ())),
                preferred_element_type=f32,
                precision=jax.lax.Precision.HIGHEST)
        zn = (_dot(nf, nW0x_ref[l]) + _bf(aggr) * _bf(nW0a_ref[l])
              + nb0_ref[l])
        n0 = jnp.maximum(zn, 0.0)
        n1 = jnp.maximum(_dot(n0, nW1_ref[l]) + nb1_ref[l], 0.0)
        nf = nf + _dot(n1, nW2_ref[l]) + nb2_ref[l]
    out_nf_ref[0] = nf


def _full(shape):
    nd = len(shape)
    return pl.BlockSpec(shape, lambda b: (0,) * nd)


@jax.jit
def kernel(locs, W_init, b_init, eW0, eb0, eW1, eb1, eW2, eb2,
           nW0, nb0, nW1, nb1, nW2, nb2):
    eW0s = eW0[:, :D, :]
    eW0d = eW0[:, D:2 * D, :]
    eW0e = eW0[:, 2 * D, :]
    eW2r = eW2[:, :, 0]
    nW0x = nW0[:, :D, :]
    nW0a = nW0[:, D, :]
    bi = b_init.reshape(1, D)

    args = (locs, W_init, bi, eW0s, eW0d, eW0e, eb0, eW1, eb1, eW2r, eb2,
            nW0x, nW0a, nb0, nW1, nb1, nW2, nb2)
    in_specs = [pl.BlockSpec((1, N, 2), lambda b: (b, 0, 0))]
    in_specs += [_full(a.shape) for a in args[1:]]
    out_nf, out_h = pl.pallas_call(
        _body,
        grid=(B,),
        in_specs=in_specs,
        out_specs=[pl.BlockSpec((1, N, D), lambda b: (b, 0, 0))] * 2,
        out_shape=[jax.ShapeDtypeStruct((B, N, D), jnp.float32)] * 2,
        scratch_shapes=[
            pltpu.VMEM((N, N), jnp.float32),
            pltpu.VMEM((N, N), jnp.float32),
            pltpu.VMEM((N, H1), jnp.float32),
            pltpu.VMEM((N, H1), jnp.float32),
        ],
        compiler_params=pltpu.CompilerParams(
            dimension_semantics=("parallel",)),
        interpret=_INTERPRET,
    )(*args)
    return out_nf, out_h


# speedup vs baseline: 35.6641x; 1.0268x over previous
import jax
import jax.numpy as jnp
from jax.experimental import pallas as pl
from jax.experimental.pallas import tpu as pltpu

B, N, D = 4, 256, 128
H1, H2 = 64, 32
L = 3
TI = 64
NB = N // TI

_INTERPRET = False


def _dot(a, b):
    return jnp.dot(a, b, preferred_element_type=jnp.float32)


def _hdot(a, b):
    return jnp.dot(a, b, preferred_element_type=jnp.float32,
                   precision=jax.lax.Precision.HIGHEST)


def _bf(x):
    return x.astype(jnp.bfloat16).astype(jnp.float32)


def _body(locs_ref, Wi_ref, bi_ref,
          eW0s_ref, eW0d_ref, eW0e_ref, eb0_ref, eW1_ref, eb1_ref,
          eW2r_ref, eb2_ref,
          nW0x_ref, nW0a_ref, nb0_ref, nW1_ref, nb1_ref, nW2_ref, nb2_ref,
          out_nf_ref, out_h_ref,
          efs_ref, uefs_ref, As_ref, Bs_ref):
    f32 = jnp.float32
    locs = locs_ref[0]
    Wi = Wi_ref[:]
    h = _dot(locs, Wi) + bi_ref[0]
    out_h_ref[0] = h

    G = jax.lax.dot_general(h, h, (((1,), (1,)), ((), ())),
                            preferred_element_type=f32,
                            precision=jax.lax.Precision.HIGHEST)
    ii = jax.lax.broadcasted_iota(jnp.int32, (N, N), 0)
    jj = jax.lax.broadcasted_iota(jnp.int32, (N, N), 1)
    eye = (ii == jj).astype(f32)
    Gd = G * eye
    sqi = jnp.sum(Gd, axis=1, keepdims=True)
    sqj = jnp.sum(Gd, axis=0, keepdims=True)
    efs_ref[...] = jnp.sqrt(jnp.maximum(sqi + sqj - 2.0 * G, 0.0))

    ones_col = jnp.ones((N, 1), dtype=f32)
    nf = h
    for l in range(L):
        src_ref = efs_ref if l % 2 == 0 else uefs_ref
        dst_ref = uefs_ref if l % 2 == 0 else efs_ref
        As_ref[...] = _dot(nf, eW0s_ref[l]) + eb0_ref[l]
        Bs_ref[...] = _dot(nf, eW0d_ref[l])
        w0e = eW0e_ref[l]
        W1 = eW1_ref[l]
        b1 = eb1_ref[l]
        w2 = eW2r_ref[l]
        b2 = eb2_ref[l]
        last = (l == L - 1)

        def tile_body(ib, S):
            off = ib * TI
            a = As_ref[pl.ds(off, TI), :]
            e = src_ref[pl.ds(off, TI), :]
            z = (a[:, None, :] + Bs_ref[...][None, :, :]
                 + _bf(e)[:, :, None] * _bf(w0e))
            h0 = jnp.maximum(z, 0.0)
            h1 = jnp.maximum(
                _dot(h0.reshape(TI * N, H1), W1) + b1, 0.0)
            h1t = _bf(h1.reshape(TI, N, H2))
            if last:
                return S + jnp.sum(h1t, axis=0)
            u = jnp.sum(h1t * _bf(w2), axis=-1) + b2
            dst_ref[pl.ds(off, TI), :] = u
            return S

        S0 = jnp.zeros((N, H2), dtype=f32)
        S = jax.lax.fori_loop(0, NB, tile_body, S0)
        if last:
            aggr = _hdot(S, _bf(w2)[:, None]) + jnp.float32(N) * b2
        else:
            uef = dst_ref[...]
            aggr = jax.lax.dot_general(
                uef, ones_col, (((0,), (0,)), ((), ())),
                preferred_element_type=f32,
                precision=jax.lax.Precision.HIGHEST)
        zn = (_dot(nf, nW0x_ref[l]) + _bf(aggr) * _bf(nW0a_ref[l])
              + nb0_ref[l])
        n0 = jnp.maximum(zn, 0.0)
        n1 = jnp.maximum(_dot(n0, nW1_ref[l]) + nb1_ref[l], 0.0)
        nf = nf + _dot(n1, nW2_ref[l]) + nb2_ref[l]
    out_nf_ref[0] = nf


def _full(shape):
    nd = len(shape)
    return pl.BlockSpec(shape, lambda b: (0,) * nd)


@jax.jit
def kernel(locs, W_init, b_init, eW0, eb0, eW1, eb1, eW2, eb2,
           nW0, nb0, nW1, nb1, nW2, nb2):
    eW0s = eW0[:, :D, :]
    eW0d = eW0[:, D:2 * D, :]
    eW0e = eW0[:, 2 * D, :]
    eW2r = eW2[:, :, 0]
    nW0x = nW0[:, :D, :]
    nW0a = nW0[:, D, :]
    bi = b_init.reshape(1, D)

    args = (locs, W_init, bi, eW0s, eW0d, eW0e, eb0, eW1, eb1, eW2r, eb2,
            nW0x, nW0a, nb0, nW1, nb1, nW2, nb2)
    in_specs = [pl.BlockSpec((1, N, 2), lambda b: (b, 0, 0))]
    in_specs += [_full(a.shape) for a in args[1:]]
    out_nf, out_h = pl.pallas_call(
        _body,
        grid=(B,),
        in_specs=in_specs,
        out_specs=[pl.BlockSpec((1, N, D), lambda b: (b, 0, 0))] * 2,
        out_shape=[jax.ShapeDtypeStruct((B, N, D), jnp.float32)] * 2,
        scratch_shapes=[
            pltpu.VMEM((N, N), jnp.float32),
            pltpu.VMEM((N, N), jnp.float32),
            pltpu.VMEM((N, H1), jnp.float32),
            pltpu.VMEM((N, H1), jnp.float32),
        ],
        compiler_params=pltpu.CompilerParams(
            dimension_semantics=("parallel",)),
        interpret=_INTERPRET,
    )(*args)
    return out_nf, out_h


# speedup vs baseline: 46.4429x; 1.3022x over previous
import jax
import jax.numpy as jnp
from jax.experimental import pallas as pl
from jax.experimental.pallas import tpu as pltpu

B, N, D = 4, 256, 128
H1, H2 = 64, 32
L = 3
TI = 64
NB = N // TI

_INTERPRET = False


def _dot(a, b):
    return jnp.dot(a, b, preferred_element_type=jnp.float32)


def _hdot(a, b):
    return jnp.dot(a, b, preferred_element_type=jnp.float32,
                   precision=jax.lax.Precision.HIGHEST)


def _bf(x):
    return x.astype(jnp.bfloat16).astype(jnp.float32)


def _body(locs_ref, Wi_ref, bi_ref,
          eW0s_ref, eW0d_ref, eW0e_ref, eb0_ref, eW1_ref, eb1_ref,
          eW2r_ref, eb2_ref, Ms_ref, Cs_ref,
          nW0x_ref, nW0a_ref, nb0_ref, nW1_ref, nb1_ref, nW2_ref, nb2_ref,
          out_nf_ref, out_h_ref,
          efs_ref, h1s_ref, As_ref, Bs_ref):
    f32 = jnp.float32
    locs = locs_ref[0]
    Wi = Wi_ref[:]
    h = _dot(locs, Wi) + bi_ref[0]
    out_h_ref[0] = h

    G = jax.lax.dot_general(h, h, (((1,), (1,)), ((), ())),
                            preferred_element_type=f32,
                            precision=jax.lax.Precision.HIGHEST)
    ii = jax.lax.broadcasted_iota(jnp.int32, (N, N), 0)
    jj = jax.lax.broadcasted_iota(jnp.int32, (N, N), 1)
    eye = (ii == jj).astype(f32)
    Gd = G * eye
    sqi = jnp.sum(Gd, axis=1, keepdims=True)
    sqj = jnp.sum(Gd, axis=0, keepdims=True)
    efs_ref[...] = jnp.sqrt(jnp.maximum(sqi + sqj - 2.0 * G, 0.0))

    nf = h
    for l in range(L):
        A2 = _dot(nf, eW0s_ref[l]) + eb0_ref[l]
        if l > 0:
            A2 = A2 + Cs_ref[l - 1]
        As_ref[...] = A2
        Bs_ref[...] = _dot(nf, eW0d_ref[l])
        w0e = eW0e_ref[l]
        W1 = eW1_ref[l]
        b1 = eb1_ref[l]
        w2 = eW2r_ref[l]
        b2 = eb2_ref[l]
        last = (l == L - 1)

        def tile_body(ib, S):
            off = ib * TI
            eoff = ib * (TI * N)
            a = As_ref[pl.ds(off, TI), :]
            if l == 0:
                e = efs_ref[pl.ds(off, TI), :]
                c3 = _bf(e)[:, :, None] * _bf(w0e)
            else:
                hp = h1s_ref[pl.ds(eoff, TI * N), :]
                c3 = _dot(hp, Ms_ref[l - 1]).reshape(TI, N, H1)
            z = a[:, None, :] + Bs_ref[...][None, :, :] + c3
            h0 = jnp.maximum(z, 0.0)
            h1 = jnp.maximum(
                _dot(h0.reshape(TI * N, H1), W1) + b1, 0.0)
            h1b = h1.astype(jnp.bfloat16)
            if not last:
                h1s_ref[pl.ds(eoff, TI * N), :] = h1b
            return S + jnp.sum(h1b.reshape(TI, N, H2), axis=0,
                               dtype=f32)

        S = jax.lax.fori_loop(0, NB, tile_body, jnp.zeros((N, H2), f32))
        aggr = _hdot(S, _bf(w2)[:, None]) + jnp.float32(N) * b2
        zn = (_dot(nf, nW0x_ref[l]) + _bf(aggr) * _bf(nW0a_ref[l])
              + nb0_ref[l])
        n0 = jnp.maximum(zn, 0.0)
        n1 = jnp.maximum(_dot(n0, nW1_ref[l]) + nb1_ref[l], 0.0)
        nf = nf + _dot(n1, nW2_ref[l]) + nb2_ref[l]
    out_nf_ref[0] = nf


def _full(shape):
    nd = len(shape)
    return pl.BlockSpec(shape, lambda b: (0,) * nd)


@jax.jit
def kernel(locs, W_init, b_init, eW0, eb0, eW1, eb1, eW2, eb2,
           nW0, nb0, nW1, nb1, nW2, nb2):
    eW0s = eW0[:, :D, :]
    eW0d = eW0[:, D:2 * D, :]
    eW0e = eW0[:, 2 * D, :]
    eW2r = eW2[:, :, 0]
    nW0x = nW0[:, :D, :]
    nW0a = nW0[:, D, :]
    bi = b_init.reshape(1, D)
    Ms = eW2[:L - 1, :, 0:1] * eW0e[1:, None, :]
    Cs = eb2[:L - 1, 0:1] * eW0e[1:]

    args = (locs, W_init, bi, eW0s, eW0d, eW0e, eb0, eW1, eb1, eW2r, eb2,
            Ms, Cs, nW0x, nW0a, nb0, nW1, nb1, nW2, nb2)
    in_specs = [pl.BlockSpec((1, N, 2), lambda b: (b, 0, 0))]
    in_specs += [_full(a.shape) for a in args[1:]]
    out_nf, out_h = pl.pallas_call(
        _body,
        grid=(B,),
        in_specs=in_specs,
        out_specs=[pl.BlockSpec((1, N, D), lambda b: (b, 0, 0))] * 2,
        out_shape=[jax.ShapeDtypeStruct((B, N, D), jnp.float32)] * 2,
        scratch_shapes=[
            pltpu.VMEM((N, N), jnp.float32),
            pltpu.VMEM((N * N, H2), jnp.bfloat16),
            pltpu.VMEM((N, H1), jnp.float32),
            pltpu.VMEM((N, H1), jnp.float32),
        ],
        compiler_params=pltpu.CompilerParams(
            dimension_semantics=("parallel",)),
        interpret=_INTERPRET,
    )(*args)
    return out_nf, out_h


# speedup vs baseline: 50.7336x; 1.0924x over previous
import jax
import jax.numpy as jnp
from jax.experimental import pallas as pl
from jax.experimental.pallas import tpu as pltpu

B, N, D = 4, 256, 128
H1, H2 = 64, 32
L = 3
TI = 64
NB = N // TI

_INTERPRET = False


def _dot(a, b):
    return jnp.dot(a, b, preferred_element_type=jnp.float32)


def _hdot(a, b):
    return jnp.dot(a, b, preferred_element_type=jnp.float32,
                   precision=jax.lax.Precision.HIGHEST)


def _bf(x):
    return x.astype(jnp.bfloat16).astype(jnp.float32)


def _body(locs_ref, Wi_ref, bi_ref,
          eW0s_ref, eW0d_ref, eW0e_ref, eb0_ref, eW1_ref, eb1_ref,
          eW2r_ref, eb2_ref, Ms_ref, Cs_ref,
          nW0x_ref, nW0a_ref, nb0_ref, nW1_ref, nb1_ref, nW2_ref, nb2_ref,
          out_nf_ref, out_h_ref,
          efs_ref, h1s_ref, As_ref, Bs_ref):
    f32 = jnp.float32
    locs = locs_ref[0]
    Wi = Wi_ref[:]
    h = _dot(locs, Wi) + bi_ref[0]
    out_h_ref[0] = h

    G = jax.lax.dot_general(h, h, (((1,), (1,)), ((), ())),
                            preferred_element_type=f32,
                            precision=jax.lax.Precision.HIGHEST)
    ii = jax.lax.broadcasted_iota(jnp.int32, (N, N), 0)
    jj = jax.lax.broadcasted_iota(jnp.int32, (N, N), 1)
    eye = (ii == jj).astype(f32)
    Gd = G * eye
    sqi = jnp.sum(Gd, axis=1, keepdims=True)
    sqj = jnp.sum(Gd, axis=0, keepdims=True)
    efs_ref[...] = jnp.sqrt(jnp.maximum(sqi + sqj - 2.0 * G, 0.0))

    nf = h
    for l in range(L):
        A2 = _dot(nf, eW0s_ref[l]) + eb0_ref[l]
        if l > 0:
            A2 = A2 + Cs_ref[l - 1]
        As_ref[...] = A2
        Bs_ref[...] = _dot(nf, eW0d_ref[l])
        w0e = eW0e_ref[l]
        W1 = eW1_ref[l]
        b1 = eb1_ref[l]
        w2 = eW2r_ref[l]
        b2 = eb2_ref[l]
        last = (l == L - 1)

        Bfull = Bs_ref[...]

        def tile_body(ib, S):
            off = ib * TI
            eoff = ib * (TI * N)
            a = As_ref[pl.ds(off, TI), :]
            if l == 0:
                e = efs_ref[pl.ds(off, TI), :]
                c3 = _bf(e)[:, :, None] * _bf(w0e)
            else:
                hp = h1s_ref[pl.ds(eoff, TI * N), :]
                c3 = _dot(hp, Ms_ref[l - 1]).reshape(TI, N, H1)
            z = a[:, None, :] + Bfull[None, :, :] + c3
            h0 = jnp.maximum(z, 0.0)
            h1 = jnp.maximum(
                _dot(h0.reshape(TI * N, H1), W1) + b1, 0.0)
            if not last:
                h1s_ref[pl.ds(eoff, TI * N), :] = h1.astype(jnp.bfloat16)
            return S + jnp.sum(h1.reshape(TI, N, H2), axis=0)

        S = jax.lax.fori_loop(0, NB, tile_body, jnp.zeros((N, H2), f32))
        aggr = _hdot(S, _bf(w2)[:, None]) + jnp.float32(N) * b2
        zn = (_dot(nf, nW0x_ref[l]) + _bf(aggr) * _bf(nW0a_ref[l])
              + nb0_ref[l])
        n0 = jnp.maximum(zn, 0.0)
        n1 = jnp.maximum(_dot(n0, nW1_ref[l]) + nb1_ref[l], 0.0)
        nf = nf + _dot(n1, nW2_ref[l]) + nb2_ref[l]
    out_nf_ref[0] = nf


def _full(shape):
    nd = len(shape)
    return pl.BlockSpec(shape, lambda b: (0,) * nd)


@jax.jit
def kernel(locs, W_init, b_init, eW0, eb0, eW1, eb1, eW2, eb2,
           nW0, nb0, nW1, nb1, nW2, nb2):
    eW0s = eW0[:, :D, :]
    eW0d = eW0[:, D:2 * D, :]
    eW0e = eW0[:, 2 * D, :]
    eW2r = eW2[:, :, 0]
    nW0x = nW0[:, :D, :]
    nW0a = nW0[:, D, :]
    bi = b_init.reshape(1, D)
    Ms = eW2[:L - 1, :, 0:1] * eW0e[1:, None, :]
    Cs = eb2[:L - 1, 0:1] * eW0e[1:]

    args = (locs, W_init, bi, eW0s, eW0d, eW0e, eb0, eW1, eb1, eW2r, eb2,
            Ms, Cs, nW0x, nW0a, nb0, nW1, nb1, nW2, nb2)
    in_specs = [pl.BlockSpec((1, N, 2), lambda b: (b, 0, 0))]
    in_specs += [_full(a.shape) for a in args[1:]]
    out_nf, out_h = pl.pallas_call(
        _body,
        grid=(B,),
        in_specs=in_specs,
        out_specs=[pl.BlockSpec((1, N, D), lambda b: (b, 0, 0))] * 2,
        out_shape=[jax.ShapeDtypeStruct((B, N, D), jnp.float32)] * 2,
        scratch_shapes=[
            pltpu.VMEM((N, N), jnp.float32),
            pltpu.VMEM((N * N, H2), jnp.bfloat16),
            pltpu.VMEM((N, H1), jnp.float32),
            pltpu.VMEM((N, H1), jnp.float32),
        ],
        compiler_params=pltpu.CompilerParams(
            dimension_semantics=("parallel",)),
        interpret=_INTERPRET,
    )(*args)
    return out_nf, out_h
